# baseline (speedup 1.0000x reference)
import jax
import jax.numpy as jnp
from jax import lax
from jax.experimental import pallas as pl
from jax.experimental.pallas import tpu as pltpu
from jax.experimental.pallas import tpu_sc as plsc

B = 16384
S = 50
D = 32
NC = 2
NS = 16
NW = NC * NS
PW = B // NW
CB = 32
NCHUNK = PW // CB
NPAIR = NCHUNK // 2
L = 16
SP = 56


def _body(ids_hbm, table_hbm, out_hbm,
          idx0, idx1, rows0, rows1, acc0, acc1, sem0, sem1):
    wid = lax.axis_index("s") * NC + lax.axis_index("c")
    inv = jnp.float32(1.0 / S)

    def stage_and_fire(c, idx_v, rows_v, sem):
        base_rows = wid * PW + c * CB
        pltpu.sync_copy(ids_hbm.at[pl.ds(base_rows, CB), :], idx_v)
        for g in range(CB):
            pltpu.async_copy(
                table_hbm.at[idx_v.at[g, pl.ds(0, SP)]], rows_v.at[g], sem)

    def pool(c, idx_v, rows_v, acc_v, sem):
        for g in range(CB):
            pltpu.make_async_copy(
                table_hbm.at[idx_v.at[g, pl.ds(0, SP)]], rows_v.at[g],
                sem).wait()

        def row_body(g, carry2):
            a0 = rows_v[g, 0, pl.ds(0, L)]
            a1 = rows_v[g, 0, pl.ds(L, L)]
            for s in range(1, S):
                a0 = a0 + rows_v[g, s, pl.ds(0, L)]
                a1 = a1 + rows_v[g, s, pl.ds(L, L)]
            acc_v[g, pl.ds(0, L)] = a0 * inv
            acc_v[g, pl.ds(L, L)] = a1 * inv
            return carry2

        lax.fori_loop(0, CB, row_body, 0)
        base_rows = wid * PW + c * CB
        pltpu.sync_copy(acc_v, out_hbm.at[pl.ds(base_rows, CB)])

    stage_and_fire(0, idx0, rows0, sem0)

    def pair_body(i, carry):
        c0 = 2 * i
        stage_and_fire(c0 + 1, idx1, rows1, sem1)
        pool(c0, idx0, rows0, acc0, sem0)

        @pl.when(i + 1 < NPAIR)
        def _():
            stage_and_fire(c0 + 2, idx0, rows0, sem0)

        pool(c0 + 1, idx1, rows1, acc1, sem1)
        return carry

    lax.fori_loop(0, NPAIR, pair_body, 0)


@jax.jit
def _encode(ids, table):
    mesh = plsc.VectorSubcoreMesh(core_axis_name="c", subcore_axis_name="s")
    return pl.kernel(
        _body,
        out_type=jax.ShapeDtypeStruct((B, D), jnp.float32),
        mesh=mesh,
        scratch_types=[
            pltpu.VMEM((CB, 128), jnp.int32),
            pltpu.VMEM((CB, 128), jnp.int32),
            pltpu.VMEM((CB, SP, D), jnp.float32),
            pltpu.VMEM((CB, SP, D), jnp.float32),
            pltpu.VMEM((CB, D), jnp.float32),
            pltpu.VMEM((CB, D), jnp.float32),
            pltpu.SemaphoreType.DMA,
            pltpu.SemaphoreType.DMA,
        ],
        compiler_params=pltpu.CompilerParams(use_tc_tiling_on_sc=False),
    )(ids, table)


def kernel(prompt_ids, table):
    ids_p = jnp.pad(prompt_ids.astype(jnp.int32), ((0, 0), (0, 128 - S)))
    return _encode(ids_p, table)


# speedup vs baseline: 1.7064x; 1.7064x over previous
import jax
import jax.numpy as jnp
from jax import lax
from jax.experimental import pallas as pl
from jax.experimental.pallas import tpu as pltpu
from jax.experimental.pallas import tpu_sc as plsc

B = 16384
S = 50
D = 32
NC = 2
NS = 16
NW = NC * NS
PW = B // NW
CB = 32
NCHUNK = PW // CB
NPAIR = NCHUNK // 2
L = 16
FLAT = CB * S
FPAD = 1664
NROW = B // CB
GROUP = 8
NG = CB // GROUP
NV = GROUP * S // L


def _pack_body(ids_hbm, out_hbm, stage_v, flat_v):
    wid = lax.axis_index("s") * NC + lax.axis_index("c")
    iota = lax.iota(jnp.int32, L)

    def row_body(jj, carry):
        j = wid * NCHUNK + jj
        pltpu.sync_copy(ids_hbm.at[pl.ds(j * CB, CB), :], stage_v)
        for k in range(NG):
            for t in range(NV):
                q0 = 16 * t
                r0, c0 = q0 // S, q0 % S
                cut = S - c0
                if cut >= L:
                    r = jnp.full((L,), r0 + GROUP * k, jnp.int32)
                    c = iota + c0
                else:
                    ge = (iota >= cut).astype(jnp.int32)
                    r = ge + (r0 + GROUP * k)
                    c = iota + c0 - S * ge
                v = plsc.load_gather(stage_v, [r, c])
                flat_v[pl.ds(k * GROUP * S + 16 * t, L)] = v
        zero = jnp.zeros((L,), jnp.int32)
        for z in range(FLAT, FPAD, L):
            flat_v[pl.ds(z, L)] = zero
        pltpu.sync_copy(flat_v, out_hbm.at[j])
        return carry

    lax.fori_loop(0, NCHUNK, row_body, 0)


def _pool_body(ids2_hbm, table_hbm, out_hbm,
               idx0, idx1, rows0, rows1, acc0, acc1, sem0, sem1):
    wid = lax.axis_index("s") * NC + lax.axis_index("c")
    inv = jnp.float32(1.0 / S)

    def stage_and_fire(c, idx_v, rows_v, sem):
        j = wid * NCHUNK + c
        pltpu.sync_copy(ids2_hbm.at[j], idx_v)
        pltpu.async_copy(table_hbm.at[idx_v], rows_v, sem)

    def pool(c, idx_v, rows_v, acc_v, sem):
        pltpu.make_async_copy(table_hbm.at[idx_v], rows_v, sem).wait()

        def row_body(g, carry2):
            r = g * S
            a0 = rows_v[r, pl.ds(0, L)]
            a1 = rows_v[r, pl.ds(L, L)]
            for s in range(1, S):
                a0 = a0 + rows_v[r + s, pl.ds(0, L)]
                a1 = a1 + rows_v[r + s, pl.ds(L, L)]
            acc_v[g, pl.ds(0, L)] = a0 * inv
            acc_v[g, pl.ds(L, L)] = a1 * inv
            return carry2

        lax.fori_loop(0, CB, row_body, 0)
        base_rows = wid * PW + c * CB
        pltpu.sync_copy(acc_v, out_hbm.at[pl.ds(base_rows, CB)])

    stage_and_fire(0, idx0, rows0, sem0)

    def pair_body(i, carry):
        c0 = 2 * i
        stage_and_fire(c0 + 1, idx1, rows1, sem1)
        pool(c0, idx0, rows0, acc0, sem0)

        @pl.when(i + 1 < NPAIR)
        def _():
            stage_and_fire(c0 + 2, idx0, rows0, sem0)

        pool(c0 + 1, idx1, rows1, acc1, sem1)
        return carry

    lax.fori_loop(0, NPAIR, pair_body, 0)


@jax.jit
def _encode(ids, table):
    mesh = plsc.VectorSubcoreMesh(core_axis_name="c", subcore_axis_name="s")
    ids2 = pl.kernel(
        _pack_body,
        out_type=jax.ShapeDtypeStruct((NROW, FPAD), jnp.int32),
        mesh=mesh,
        scratch_types=[
            pltpu.VMEM((CB, S), jnp.int32),
            pltpu.VMEM((FPAD,), jnp.int32),
        ],
        compiler_params=pltpu.CompilerParams(
            use_tc_tiling_on_sc=True, needs_layout_passes=False),
    )(ids)
    return pl.kernel(
        _pool_body,
        out_type=jax.ShapeDtypeStruct((B, D), jnp.float32),
        mesh=mesh,
        scratch_types=[
            pltpu.VMEM((FPAD,), jnp.int32),
            pltpu.VMEM((FPAD,), jnp.int32),
            pltpu.VMEM((FPAD, D), jnp.float32),
            pltpu.VMEM((FPAD, D), jnp.float32),
            pltpu.VMEM((CB, D), jnp.float32),
            pltpu.VMEM((CB, D), jnp.float32),
            pltpu.SemaphoreType.DMA,
            pltpu.SemaphoreType.DMA,
        ],
        compiler_params=pltpu.CompilerParams(use_tc_tiling_on_sc=False),
    )(ids2, table)


def kernel(prompt_ids, table):
    return _encode(prompt_ids.astype(jnp.int32), table)


# speedup vs baseline: 1.7238x; 1.0102x over previous
import jax
import jax.numpy as jnp
from jax import lax
from jax.experimental import pallas as pl
from jax.experimental.pallas import tpu as pltpu
from jax.experimental.pallas import tpu_sc as plsc

B = 16384
S = 50
D = 32
NC = 2
NS = 16
NW = NC * NS
PW = B // NW
CB = 32
NCHUNK = PW // CB
NPAIR = NCHUNK // 2
L = 16
FLAT = CB * S
FPAD = 1664
NROW = B // CB
GROUP = 8
NG = CB // GROUP
NV = GROUP * S // L


def _pack_body(ids_hbm, out_hbm, stage_v, flat_v):
    wid = lax.axis_index("s") * NC + lax.axis_index("c")
    iota = lax.iota(jnp.int32, L)

    def row_body(jj, carry):
        j = wid * NCHUNK + jj
        pltpu.sync_copy(ids_hbm.at[pl.ds(j * CB, CB), :], stage_v)
        for k in range(NG):
            for t in range(NV):
                q0 = 16 * t
                r0, c0 = q0 // S, q0 % S
                cut = S - c0
                if cut >= L:
                    r = jnp.full((L,), r0 + GROUP * k, jnp.int32)
                    c = iota + c0
                else:
                    ge = (iota >= cut).astype(jnp.int32)
                    r = ge + (r0 + GROUP * k)
                    c = iota + c0 - S * ge
                v = plsc.load_gather(stage_v, [r, c])
                flat_v[pl.ds(k * GROUP * S + 16 * t, L)] = v
        zero = jnp.zeros((L,), jnp.int32)
        for z in range(FLAT, FPAD, L):
            flat_v[pl.ds(z, L)] = zero
        pltpu.sync_copy(flat_v, out_hbm.at[pl.ds(j * FPAD, FPAD)])
        return carry

    lax.fori_loop(0, NCHUNK, row_body, 0)


def _pool_body(ids2_hbm, table_hbm, out_hbm,
               idx0, idx1, rows0, rows1, acc0, acc1, sem0, sem1):
    wid = lax.axis_index("s") * NC + lax.axis_index("c")
    inv = jnp.float32(1.0 / S)

    def stage_and_fire(c, idx_v, rows_v, sem):
        j = wid * NCHUNK + c
        pltpu.sync_copy(ids2_hbm.at[pl.ds(j * FPAD, FPAD)], idx_v)
        pltpu.async_copy(table_hbm.at[idx_v], rows_v, sem)

    def pool(c, idx_v, rows_v, acc_v, sem):
        pltpu.make_async_copy(table_hbm.at[idx_v], rows_v, sem).wait()

        def row_body(g, carry2):
            r = g * S
            a0 = rows_v[r, pl.ds(0, L)]
            a1 = rows_v[r, pl.ds(L, L)]
            for s in range(1, S):
                a0 = a0 + rows_v[r + s, pl.ds(0, L)]
                a1 = a1 + rows_v[r + s, pl.ds(L, L)]
            acc_v[g, pl.ds(0, L)] = a0 * inv
            acc_v[g, pl.ds(L, L)] = a1 * inv
            return carry2

        lax.fori_loop(0, CB, row_body, 0)
        base_rows = wid * PW + c * CB
        pltpu.sync_copy(acc_v, out_hbm.at[pl.ds(base_rows, CB)])

    stage_and_fire(0, idx0, rows0, sem0)

    def pair_body(i, carry):
        c0 = 2 * i
        stage_and_fire(c0 + 1, idx1, rows1, sem1)
        pool(c0, idx0, rows0, acc0, sem0)

        @pl.when(i + 1 < NPAIR)
        def _():
            stage_and_fire(c0 + 2, idx0, rows0, sem0)

        pool(c0 + 1, idx1, rows1, acc1, sem1)
        return carry

    lax.fori_loop(0, NPAIR, pair_body, 0)


@jax.jit
def _encode(ids, table):
    mesh = plsc.VectorSubcoreMesh(core_axis_name="c", subcore_axis_name="s")
    ids2 = pl.kernel(
        _pack_body,
        out_type=jax.ShapeDtypeStruct((NROW * FPAD,), jnp.int32),
        mesh=mesh,
        scratch_types=[
            pltpu.VMEM((CB, S), jnp.int32),
            pltpu.VMEM((FPAD,), jnp.int32),
        ],
        compiler_params=pltpu.CompilerParams(
            use_tc_tiling_on_sc=True, needs_layout_passes=False),
    )(ids)
    return pl.kernel(
        _pool_body,
        out_type=jax.ShapeDtypeStruct((B, D), jnp.float32),
        mesh=mesh,
        scratch_types=[
            pltpu.VMEM((FPAD,), jnp.int32),
            pltpu.VMEM((FPAD,), jnp.int32),
            pltpu.VMEM((FPAD, D), jnp.float32),
            pltpu.VMEM((FPAD, D), jnp.float32),
            pltpu.VMEM((CB, D), jnp.float32),
            pltpu.VMEM((CB, D), jnp.float32),
            pltpu.SemaphoreType.DMA,
            pltpu.SemaphoreType.DMA,
        ],
        compiler_params=pltpu.CompilerParams(use_tc_tiling_on_sc=False),
    )(ids2, table)


def kernel(prompt_ids, table):
    return _encode(prompt_ids.astype(jnp.int32), table)


# speedup vs baseline: 2.5932x; 1.5044x over previous
import jax
import jax.numpy as jnp
from jax import lax
from jax.experimental import pallas as pl
from jax.experimental.pallas import tpu as pltpu
from jax.experimental.pallas import tpu_sc as plsc

B = 16384
S = 50
D = 32
NC = 2
NS = 16
NW = NC * NS
PW = B // NW
CB = 32
NCHUNK = PW // CB
NPAIR = NCHUNK // 2
L = 16
FLAT = CB * S
FPAD = 1664
NROW = B // CB
GROUP = 8
NG = CB // GROUP
NV = GROUP * S // L


def _pack_body(ids_hbm, out_hbm, stage_v, flat_v):
    wid = lax.axis_index("s") * NC + lax.axis_index("c")
    iota = lax.iota(jnp.int32, L)

    def row_body(jj, carry):
        j = wid * NCHUNK + jj
        pltpu.sync_copy(ids_hbm.at[pl.ds(j * CB, CB), :], stage_v)
        for k in range(NG):
            for t in range(NV):
                q0 = 16 * t
                r0, c0 = q0 // S, q0 % S
                cut = S - c0
                if cut >= L:
                    r = jnp.full((L,), r0 + GROUP * k, jnp.int32)
                    c = iota + c0
                else:
                    ge = (iota >= cut).astype(jnp.int32)
                    r = ge + (r0 + GROUP * k)
                    c = iota + c0 - S * ge
                v = plsc.load_gather(stage_v, [r, c])
                flat_v[pl.ds(k * GROUP * S + 16 * t, L)] = v
        for z in range(FLAT, FPAD, L):
            flat_v[pl.ds(z, L)] = iota + z
        pltpu.sync_copy(flat_v, out_hbm.at[pl.ds(j * FPAD, FPAD)])
        return carry

    lax.fori_loop(0, NCHUNK, row_body, 0)


def _pool_body(ids2_hbm, table_hbm, out_hbm,
               idx0, idx1, rows0, rows1, acc0, acc1, sem0, sem1):
    wid = lax.axis_index("s") * NC + lax.axis_index("c")
    inv = jnp.float32(1.0 / S)

    def stage_and_fire(c, idx_v, rows_v, sem):
        j = wid * NCHUNK + c
        pltpu.sync_copy(ids2_hbm.at[pl.ds(j * FPAD, FPAD)], idx_v)
        pltpu.async_copy(table_hbm.at[idx_v], rows_v, sem)

    def pool(c, idx_v, rows_v, acc_v, sem):
        pltpu.make_async_copy(table_hbm.at[idx_v], rows_v, sem).wait()

        def row_body(g, carry2):
            r = g * S
            a0 = rows_v[r, pl.ds(0, L)]
            a1 = rows_v[r, pl.ds(L, L)]
            for s in range(1, S):
                a0 = a0 + rows_v[r + s, pl.ds(0, L)]
                a1 = a1 + rows_v[r + s, pl.ds(L, L)]
            acc_v[g, pl.ds(0, L)] = a0 * inv
            acc_v[g, pl.ds(L, L)] = a1 * inv
            return carry2

        lax.fori_loop(0, CB, row_body, 0)
        base_rows = wid * PW + c * CB
        pltpu.sync_copy(acc_v, out_hbm.at[pl.ds(base_rows, CB)])

    stage_and_fire(0, idx0, rows0, sem0)

    def pair_body(i, carry):
        c0 = 2 * i
        stage_and_fire(c0 + 1, idx1, rows1, sem1)
        pool(c0, idx0, rows0, acc0, sem0)

        @pl.when(i + 1 < NPAIR)
        def _():
            stage_and_fire(c0 + 2, idx0, rows0, sem0)

        pool(c0 + 1, idx1, rows1, acc1, sem1)
        return carry

    lax.fori_loop(0, NPAIR, pair_body, 0)


@jax.jit
def _encode(ids, table):
    mesh = plsc.VectorSubcoreMesh(core_axis_name="c", subcore_axis_name="s")
    ids2 = pl.kernel(
        _pack_body,
        out_type=jax.ShapeDtypeStruct((NROW * FPAD,), jnp.int32),
        mesh=mesh,
        scratch_types=[
            pltpu.VMEM((CB, S), jnp.int32),
            pltpu.VMEM((FPAD,), jnp.int32),
        ],
        compiler_params=pltpu.CompilerParams(
            use_tc_tiling_on_sc=True, needs_layout_passes=False),
    )(ids)
    return pl.kernel(
        _pool_body,
        out_type=jax.ShapeDtypeStruct((B, D), jnp.float32),
        mesh=mesh,
        scratch_types=[
            pltpu.VMEM((FPAD,), jnp.int32),
            pltpu.VMEM((FPAD,), jnp.int32),
            pltpu.VMEM((FPAD, D), jnp.float32),
            pltpu.VMEM((FPAD, D), jnp.float32),
            pltpu.VMEM((CB, D), jnp.float32),
            pltpu.VMEM((CB, D), jnp.float32),
            pltpu.SemaphoreType.DMA,
            pltpu.SemaphoreType.DMA,
        ],
        compiler_params=pltpu.CompilerParams(use_tc_tiling_on_sc=False),
    )(ids2, table)


def kernel(prompt_ids, table):
    return _encode(prompt_ids.astype(jnp.int32), table)
